# baseline (speedup 1.0000x reference)
import math

import jax
import jax.numpy as jnp
from jax import lax
from jax.experimental import pallas as pl
from jax.experimental.pallas import tpu as pltpu
from jax.experimental.pallas import tpu_sc as plsc

NC = 2
NS = 16
NW = NC * NS
LANES = 16
CHUNK = 128
NBUF = 4


def _sc_embedding_lookup(tok3, table, n_chunks, d, scale):
    total = NW * n_chunks * CHUNK
    per_w = n_chunks * CHUNK
    mesh = plsc.VectorSubcoreMesh(core_axis_name="c", subcore_axis_name="s")

    @pl.kernel(
        out_type=jax.ShapeDtypeStruct((total, d), jnp.float32),
        mesh=mesh,
        compiler_params=pltpu.CompilerParams(use_tc_tiling_on_sc=False),
        scratch_types=[
            pltpu.VMEM((n_chunks, CHUNK), jnp.int32),
            pltpu.VMEM((NBUF, CHUNK, d), jnp.float32),
            pltpu.VMEM((NBUF, CHUNK, d), jnp.float32),
            pltpu.SemaphoreType.DMA((NBUF,)),
            pltpu.SemaphoreType.DMA((NBUF,)),
        ],
    )
    def k(tok_hbm, table_hbm, out_hbm, idx_v, gbuf, sbuf, gsem, ssem):
        wid = lax.axis_index("c") * NS + lax.axis_index("s")
        row0 = wid * per_w

        pltpu.sync_copy(tok_hbm.at[wid], idx_v)

        def gather(j, b):
            return pltpu.make_async_copy(
                table_hbm.at[idx_v.at[j]], gbuf.at[b], gsem.at[b])

        def store(j, b):
            return pltpu.make_async_copy(
                sbuf.at[b], out_hbm.at[pl.ds(row0 + j * CHUNK, CHUNK)],
                ssem.at[b])

        def scale_chunk(b):
            g = gbuf.at[b]
            s = sbuf.at[b]

            @pl.loop(0, CHUNK)
            def _(r):
                for c in range(d // LANES):
                    sl = (pl.ds(r, 1), pl.ds(c * LANES, LANES))
                    s.at[sl][...] = g.at[sl][...] * scale

        def process(j, b, wait_store, issue_next):
            gather(j, b).wait()
            if wait_store:
                store(j, b).wait()
            scale_chunk(b)
            if issue_next:
                gather(j + NBUF, b).start()
            store(j, b).start()

        for b in range(NBUF):
            gather(b, b).start()
        for b in range(NBUF):
            process(b, b, wait_store=False, issue_next=True)

        @pl.loop(1, n_chunks // NBUF - 1)
        def _(g):
            j0 = g * NBUF
            for b in range(NBUF):
                process(j0 + b, b, wait_store=True, issue_next=True)

        j0 = n_chunks - NBUF
        for b in range(NBUF):
            process(j0 + b, b, wait_store=True, issue_next=False)

        for b in range(NBUF):
            store(j0 + b, b).wait()

    return k(tok3, table)


def kernel(token_ids, embedding_table):
    bsz, seq = token_ids.shape
    v, d = embedding_table.shape
    total = bsz * seq
    assert total % (NW * CHUNK) == 0 and d % LANES == 0
    n_chunks = total // (NW * CHUNK)
    scale = math.sqrt(d)
    tok3 = token_ids.astype(jnp.int32).reshape(NW, n_chunks, CHUNK)
    out = _sc_embedding_lookup(tok3, embedding_table, n_chunks, d, scale)
    return out.reshape(bsz, seq, d)


# speedup vs baseline: 1.0017x; 1.0017x over previous
import math

import jax
import jax.numpy as jnp
from jax import lax
from jax.experimental import pallas as pl
from jax.experimental.pallas import tpu as pltpu
from jax.experimental.pallas import tpu_sc as plsc

NC = 2
NS = 16
NW = NC * NS
LANES = 16
CHUNK = 128
NBUF = 4


def _sc_embedding_lookup(tok3, table, n_chunks, d, scale):
    total = NW * n_chunks * CHUNK
    per_w = n_chunks * CHUNK
    mesh = plsc.VectorSubcoreMesh(core_axis_name="c", subcore_axis_name="s")

    @pl.kernel(
        out_type=jax.ShapeDtypeStruct((total, d), jnp.float32),
        mesh=mesh,
        compiler_params=pltpu.CompilerParams(use_tc_tiling_on_sc=False),
        scratch_types=[
            pltpu.VMEM((n_chunks, CHUNK), jnp.int32),
            pltpu.VMEM((NBUF, CHUNK, d), jnp.float32),
            pltpu.VMEM((NBUF, CHUNK, d), jnp.float32),
            pltpu.SemaphoreType.DMA((NBUF,)),
            pltpu.SemaphoreType.DMA((NBUF,)),
        ],
    )
    def k(tok_hbm, table_hbm, out_hbm, idx_v, gbuf, sbuf, gsem, ssem):
        wid = lax.axis_index("c") * NS + lax.axis_index("s")
        row0 = wid * per_w

        pltpu.sync_copy(tok_hbm.at[wid], idx_v)

        def gather(j, b):
            return pltpu.make_async_copy(
                table_hbm.at[idx_v.at[j]], gbuf.at[b], gsem.at[b])

        def store(j, b):
            return pltpu.make_async_copy(
                sbuf.at[b], out_hbm.at[pl.ds(row0 + j * CHUNK, CHUNK)],
                ssem.at[b])

        def scale_chunk(b):
            g = gbuf.at[b]
            s = sbuf.at[b]

            @pl.loop(0, CHUNK, step=8)
            def _(r):
                for dr in range(8):
                    for c in range(d // LANES):
                        sl = (pl.ds(r + dr, 1), pl.ds(c * LANES, LANES))
                        s.at[sl][...] = g.at[sl][...] * scale

        def process(j, b, wait_store, issue_next):
            gather(j, b).wait()
            if wait_store:
                store(j, b).wait()
            scale_chunk(b)
            if issue_next:
                gather(j + NBUF, b).start()
            store(j, b).start()

        for b in range(NBUF):
            gather(b, b).start()
        for b in range(NBUF):
            process(b, b, wait_store=False, issue_next=True)

        @pl.loop(1, n_chunks // NBUF - 1)
        def _(g):
            j0 = g * NBUF
            for b in range(NBUF):
                process(j0 + b, b, wait_store=True, issue_next=True)

        j0 = n_chunks - NBUF
        for b in range(NBUF):
            process(j0 + b, b, wait_store=True, issue_next=False)

        for b in range(NBUF):
            store(j0 + b, b).wait()

    return k(tok3, table)


def kernel(token_ids, embedding_table):
    bsz, seq = token_ids.shape
    v, d = embedding_table.shape
    total = bsz * seq
    assert total % (NW * CHUNK) == 0 and d % LANES == 0
    n_chunks = total // (NW * CHUNK)
    scale = math.sqrt(d)
    tok3 = token_ids.astype(jnp.int32).reshape(NW, n_chunks, CHUNK)
    out = _sc_embedding_lookup(tok3, embedding_table, n_chunks, d, scale)
    return out.reshape(bsz, seq, d)
